# baseline (speedup 1.0000x reference)
import jax
import jax.numpy as jnp
from jax.experimental import pallas as pl
from jax.experimental.pallas import tpu as pltpu

N_NODES = 1000
N_FEATS = 64
N_SLABS = 5
SLAB = N_NODES // N_SLABS


def _gcn_kernel(a_hbm, x_ref, wmat_ref, b_ref, out_ref, a_vmem, sems):
    copies = [
        pltpu.make_async_copy(
            a_hbm.at[pl.ds(i * SLAB, SLAB), :],
            a_vmem.at[pl.ds(i * SLAB, SLAB), :],
            sems.at[i],
        )
        for i in range(N_SLABS)
    ]
    for c in copies:
        c.start()
    xw = jnp.dot(x_ref[...], wmat_ref[...], preferred_element_type=jnp.float32)
    for c in copies:
        c.wait()
    a = a_vmem[...]
    ones = jnp.ones((N_NODES, 1), dtype=jnp.float32)
    deg = jax.lax.dot_general(
        a, ones, (((0,), (0,)), ((), ())), preferred_element_type=jnp.float32
    )
    safe = jnp.where(deg > 0, deg, 1.0)
    dinv = jnp.where(deg > 0, jax.lax.rsqrt(safe), 0.0)
    y = dinv * xw
    agg = jax.lax.dot_general(
        a, y, (((0,), (0,)), ((), ())), preferred_element_type=jnp.float32
    )
    out_ref[...] = dinv * agg + b_ref[...].reshape(1, N_FEATS)


def kernel(input, edge_index, edge_weights, W, b):
    del edge_index
    a = edge_weights.reshape(N_NODES, N_NODES)
    return pl.pallas_call(
        _gcn_kernel,
        in_specs=[
            pl.BlockSpec(memory_space=pltpu.MemorySpace.HBM),
            pl.BlockSpec((N_NODES, N_FEATS), lambda: (0, 0)),
            pl.BlockSpec((N_FEATS, N_FEATS), lambda: (0, 0)),
            pl.BlockSpec((N_FEATS,), lambda: (0,)),
        ],
        out_specs=pl.BlockSpec((N_NODES, N_FEATS), lambda: (0, 0)),
        out_shape=jax.ShapeDtypeStruct((N_NODES, N_FEATS), jnp.float32),
        scratch_shapes=[
            pltpu.VMEM((N_NODES, N_NODES), jnp.float32),
            pltpu.SemaphoreType.DMA((N_SLABS,)),
        ],
    )(a, input, W, b)


# speedup vs baseline: 1.1990x; 1.1990x over previous
import jax
import jax.numpy as jnp
from jax.experimental import pallas as pl

N_NODES = 1000
N_FEATS = 64


def _gcn_kernel(a_ref, x_ref, wmat_ref, b_ref, out_ref):
    a = a_ref[...]
    deg = jnp.sum(a.astype(jnp.float32), axis=0, keepdims=True)
    safe = jnp.where(deg > 0, deg, 1.0)
    dinv = jnp.where(deg > 0, jax.lax.rsqrt(safe), 0.0)
    xw = jnp.dot(x_ref[...], wmat_ref[...], preferred_element_type=jnp.float32)
    xw_t = jax.lax.transpose(xw, (1, 0))
    y_t = (dinv * xw_t).astype(jnp.bfloat16)
    agg_t = jnp.dot(y_t, a, preferred_element_type=jnp.float32)
    out_t = dinv * agg_t + b_ref[...].reshape(N_FEATS, 1)
    out_ref[...] = jax.lax.transpose(out_t, (1, 0))


def kernel(input, edge_index, edge_weights, W, b):
    del edge_index
    a = edge_weights.astype(jnp.bfloat16).reshape(N_NODES, N_NODES)
    return pl.pallas_call(
        _gcn_kernel,
        out_shape=jax.ShapeDtypeStruct((N_NODES, N_FEATS), jnp.float32),
    )(a, input, W, b)
